# baseline (speedup 1.0000x reference)
import jax
import jax.numpy as jnp
from jax.experimental import pallas as pl
from jax.experimental.pallas import tpu as pltpu

_TILE = 256


def _gcn_body(a_ref, x_ref, k1_ref, k2_ref, b1_ref, b2_ref, o_ref):
    a = a_ref[...]
    bf = jnp.bfloat16
    p1 = jnp.dot(a, x_ref[...], preferred_element_type=jnp.float32)
    h1 = jnp.dot(p1.astype(bf), k1_ref[...], preferred_element_type=jnp.float32)
    h1 = h1 + b1_ref[...]
    h1 = jnp.where(h1 >= 0, h1, 0.01 * h1)
    p2 = jnp.dot(a, h1.astype(bf), preferred_element_type=jnp.float32)
    h2 = jnp.dot(p2.astype(bf), k2_ref[...], preferred_element_type=jnp.float32)
    h2 = h2 + b2_ref[...]
    o_ref[...] = jnp.where(h2 >= 0, h2, 0.01 * h2)


def kernel(X, A, W1, b1, W2, b2):
    B, N, T, F_in = X.shape
    F_sp = W1.shape[1]
    assert F_in == F_sp, "flattened-column layout assumes F_in == F_sp"
    C = B * T * F_in

    bf = jnp.bfloat16
    Xmat = jnp.transpose(X.astype(bf), (1, 0, 2, 3)).reshape(N, C)
    Abf = A.astype(bf)

    tile = min(_TILE, C)
    nblk = tile // F_in
    eye = jnp.eye(nblk, dtype=bf)
    K1 = jnp.kron(eye, W1.astype(bf))
    K2 = jnp.kron(eye, W2.astype(bf))
    b1t = jnp.tile(b1, nblk)[None, :]
    b2t = jnp.tile(b2, nblk)[None, :]

    grid = (C // tile,)
    out = pl.pallas_call(
        _gcn_body,
        grid=grid,
        in_specs=[
            pl.BlockSpec((N, N), lambda j: (0, 0)),
            pl.BlockSpec((N, tile), lambda j: (0, j)),
            pl.BlockSpec((tile, tile), lambda j: (0, 0)),
            pl.BlockSpec((tile, tile), lambda j: (0, 0)),
            pl.BlockSpec((1, tile), lambda j: (0, 0)),
            pl.BlockSpec((1, tile), lambda j: (0, 0)),
        ],
        out_specs=pl.BlockSpec((N, tile), lambda j: (0, j)),
        out_shape=jax.ShapeDtypeStruct((N, C), jnp.float32),
        compiler_params=pltpu.CompilerParams(
            dimension_semantics=("arbitrary",),
        ),
    )(Abf, Xmat, K1, K2, b1t, b2t)

    return jnp.transpose(out.reshape(N, B, T, F_sp), (1, 0, 2, 3))


# speedup vs baseline: 1.1368x; 1.1368x over previous
import jax
import jax.numpy as jnp
from jax.experimental import pallas as pl
from jax.experimental.pallas import tpu as pltpu

_TILE = 256


def _gcn_body(a_ref, x_ref, k1_ref, k2_ref, b1_ref, b2_ref, o_ref):
    a = a_ref[...]
    p1 = jnp.dot(a, x_ref[...], preferred_element_type=jnp.float32)
    h1 = jnp.dot(p1, k1_ref[...], preferred_element_type=jnp.float32)
    h1 = h1 + b1_ref[...]
    h1 = jnp.where(h1 >= 0, h1, 0.01 * h1)
    p2 = jnp.dot(a, h1, preferred_element_type=jnp.float32)
    h2 = jnp.dot(p2, k2_ref[...], preferred_element_type=jnp.float32)
    h2 = h2 + b2_ref[...]
    o_ref[...] = jnp.where(h2 >= 0, h2, 0.01 * h2)


def kernel(X, A, W1, b1, W2, b2):
    B, N, T, F_in = X.shape
    F_sp = W1.shape[1]
    assert F_in == F_sp, "flattened-column layout assumes F_in == F_sp"
    C = B * T * F_in

    Xmat = jnp.transpose(X, (1, 0, 2, 3)).reshape(N, C)

    tile = min(_TILE, C)
    nblk = tile // F_in
    eye = jnp.eye(nblk, dtype=X.dtype)
    K1 = jnp.kron(eye, W1)
    K2 = jnp.kron(eye, W2)
    b1t = jnp.tile(b1, nblk)[None, :]
    b2t = jnp.tile(b2, nblk)[None, :]

    grid = (C // tile,)
    out = pl.pallas_call(
        _gcn_body,
        grid=grid,
        in_specs=[
            pl.BlockSpec((N, N), lambda j: (0, 0)),
            pl.BlockSpec((N, tile), lambda j: (0, j)),
            pl.BlockSpec((tile, tile), lambda j: (0, 0)),
            pl.BlockSpec((tile, tile), lambda j: (0, 0)),
            pl.BlockSpec((1, tile), lambda j: (0, 0)),
            pl.BlockSpec((1, tile), lambda j: (0, 0)),
        ],
        out_specs=pl.BlockSpec((N, tile), lambda j: (0, j)),
        out_shape=jax.ShapeDtypeStruct((N, C), jnp.float32),
        compiler_params=pltpu.CompilerParams(
            dimension_semantics=("arbitrary",),
        ),
    )(A, Xmat, K1, K2, b1t, b2t)

    return jnp.transpose(out.reshape(N, B, T, F_sp), (1, 0, 2, 3))
